# baseline (speedup 1.0000x reference)
import functools

import jax
import jax.numpy as jnp
from jax import lax
from jax.experimental import pallas as pl
from jax.experimental.pallas import tpu as pltpu
from jax.experimental.pallas import tpu_sc as plsc

NUM_POSITIONS = 8192
EMBED_DIM = 2048
BATCH = 4
SEQ_LEN = 8192
N_ROWS = BATCH * SEQ_LEN

_INFO = plsc.get_sparse_core_info()
_NC = _INFO.num_cores
_NS = _INFO.num_subcores
_NW = _NC * _NS
_PW = N_ROWS // _NW

_C = 16
_NBUF = 2
_STEPS = _PW // _C


def _make_embed_kernel():
  mesh = plsc.VectorSubcoreMesh(core_axis_name="c", subcore_axis_name="s")
  scratch = [pltpu.VMEM((_PW,), jnp.int32)]
  scratch += [pltpu.VMEM((_C, EMBED_DIM), jnp.float32) for _ in range(_NBUF)]
  scratch += [pltpu.SemaphoreType.DMA for _ in range(2 * _NBUF)]

  @functools.partial(
      pl.kernel,
      mesh=mesh,
      out_type=jax.ShapeDtypeStruct((N_ROWS, EMBED_DIM), jnp.float32),
      scratch_types=scratch,
  )
  def embed(x_hbm, w_hbm, out_hbm, idx_v, *rest):
    bufs = rest[:_NBUF]
    gsems = rest[_NBUF:2 * _NBUF]
    ssems = rest[2 * _NBUF:]

    wid = lax.axis_index("s") * _NC + lax.axis_index("c")
    base = wid * _PW
    pltpu.sync_copy(x_hbm.at[pl.ds(base, _PW)], idx_v)

    def gather(slot, chunk):
      return pltpu.make_async_copy(
          w_hbm.at[idx_v.at[pl.ds(chunk * _C, _C)]], bufs[slot], gsems[slot])

    def scatter(slot, chunk):
      return pltpu.make_async_copy(
          bufs[slot], out_hbm.at[pl.ds(base + chunk * _C, _C)], ssems[slot])

    for b in range(_NBUF):
      gather(b, b).start()

    def body(i, carry):
      for b in range(_NBUF):
        gather(b, i + b).wait()
        scatter(b, i + b).start()
      for b in range(_NBUF):
        scatter(b, i + b).wait()
        gather(b, i + b + _NBUF).start()
      return carry

    lax.fori_loop(0, (_STEPS - _NBUF) // _NBUF,
                  lambda i, c: body(i * _NBUF, c), 0, unroll=False)

    last = _STEPS - _NBUF
    for b in range(_NBUF):
      gather(b, last + b).wait()
      scatter(b, last + b).start()
    for b in range(_NBUF):
      scatter(b, last + b).wait()

  return embed


_EMBED = _make_embed_kernel()


def kernel(x, weight):
  x_flat = x.reshape(N_ROWS).astype(jnp.int32)
  out = _EMBED(x_flat, weight)
  return out.reshape(BATCH, SEQ_LEN, EMBED_DIM)


# speedup vs baseline: 1.0338x; 1.0338x over previous
import functools

import jax
import jax.numpy as jnp
from jax import lax
from jax.experimental import pallas as pl
from jax.experimental.pallas import tpu as pltpu
from jax.experimental.pallas import tpu_sc as plsc

NUM_POSITIONS = 8192
EMBED_DIM = 2048
BATCH = 4
SEQ_LEN = 8192
N_ROWS = BATCH * SEQ_LEN

_INFO = plsc.get_sparse_core_info()
_NC = _INFO.num_cores
_NS = _INFO.num_subcores
_NW = _NC * _NS
_PW = N_ROWS // _NW

_C = 8
_NBUF = 6
_STEPS = _PW // _C


def _make_embed_kernel():
  mesh = plsc.VectorSubcoreMesh(core_axis_name="c", subcore_axis_name="s")
  scratch = [pltpu.VMEM((_PW,), jnp.int32)]
  scratch += [pltpu.VMEM((_C, EMBED_DIM), jnp.float32) for _ in range(_NBUF)]
  scratch += [pltpu.SemaphoreType.DMA for _ in range(2 * _NBUF)]

  @functools.partial(
      pl.kernel,
      mesh=mesh,
      out_type=jax.ShapeDtypeStruct((N_ROWS, EMBED_DIM), jnp.float32),
      scratch_types=scratch,
  )
  def embed(x_hbm, w_hbm, out_hbm, idx_v, *rest):
    bufs = rest[:_NBUF]
    gsems = rest[_NBUF:2 * _NBUF]
    ssems = rest[2 * _NBUF:]

    wid = lax.axis_index("s") * _NC + lax.axis_index("c")
    base = wid * _PW
    pltpu.sync_copy(x_hbm.at[pl.ds(base, _PW)], idx_v)

    def gather(slot, chunk):
      return pltpu.make_async_copy(
          w_hbm.at[idx_v.at[pl.ds(chunk * _C, _C)]], bufs[slot], gsems[slot])

    def scatter(slot, chunk):
      return pltpu.make_async_copy(
          bufs[slot], out_hbm.at[pl.ds(base + chunk * _C, _C)], ssems[slot])

    for b in range(_NBUF):
      gather(b, b).start()

    def body(i, carry):
      for b in range(_NBUF):
        gather(b, i + b).wait()
        scatter(b, i + b).start()
      for b in range(_NBUF):
        scatter(b, i + b).wait()
        gather(b, i + b + _NBUF).start()
      return carry

    n_main = (_STEPS - _NBUF) // _NBUF
    lax.fori_loop(0, n_main, lambda i, c: body(i * _NBUF, c), 0, unroll=False)

    done = n_main * _NBUF
    for g in range(done, _STEPS):
      b = g % _NBUF
      gather(b, g).wait()
      scatter(b, g).start()
      if g + _NBUF < _STEPS:
        scatter(b, g).wait()
        gather(b, g + _NBUF).start()
    for g in range(max(done, _STEPS - _NBUF), _STEPS):
      scatter(g % _NBUF, g).wait()

  return embed


_EMBED = _make_embed_kernel()


def kernel(x, weight):
  x_flat = x.reshape(N_ROWS).astype(jnp.int32)
  out = _EMBED(x_flat, weight)
  return out.reshape(BATCH, SEQ_LEN, EMBED_DIM)
